# Initial kernel scaffold; baseline (speedup 1.0000x reference)
import jax
import jax.numpy as jnp
from jax.experimental import pallas as pl


def kernel(x, edge_index, Wl0, bl0, Wr0, Wl1, bl1, Wr1, Wl2, bl2, Wr2):
    raise NotImplementedError("write your pallas kernel here")


# speedup vs baseline: 5.0412x; 5.0412x over previous
import functools

import jax
import jax.numpy as jnp
from jax import lax
from jax.experimental import pallas as pl
from jax.experimental.pallas import tpu as pltpu
from jax.experimental.pallas import tpu_sc as plsc

N = 10000
E = 320000
D = 128

NC = 2
NS = 16
NW = NC * NS
CHUNK = 128
NCH = -(-E // (NW * CHUNK))
EPW = NCH * CHUNK
E_PAD = EPW * NW
RPT = -(-(N + 1) // (NS * 8)) * 8
ACC_ROWS = RPT * NS
CW = 16


def _sc_body(with_counts, *refs):
    if with_counts:
        (h, srcg, dstg, zacc, zcnt, agg_out, cnt_out,
         src_v, dst_v, rows_v, cnt_priv, acc_sh, sem) = refs
    else:
        (h, srcg, dstg, zacc, agg_out,
         src_v, dst_v, rows_v, acc_sh, sem) = refs
    c = lax.axis_index("c")
    s = lax.axis_index("s")
    gwid = c * NS + s

    pltpu.sync_copy(zacc, acc_sh.at[pl.ds(s * RPT, RPT)])
    if with_counts:
        pltpu.sync_copy(zcnt, cnt_priv)
    pltpu.sync_copy(srcg.at[gwid], src_v)
    pltpu.sync_copy(dstg.at[gwid], dst_v)
    plsc.subcore_barrier()

    ones16 = jnp.ones((16,), jnp.float32)

    def step(j, carry):
        pltpu.async_copy(h.at[src_v.at[j]], rows_v, sem).wait()
        pltpu.sync_copy(rows_v, acc_sh.at[dst_v.at[j]], add=True)
        if with_counts:
            for k in range(CHUNK // 16):
                idx = dst_v[j, pl.ds(k * 16, 16)]
                plsc.addupdate_scatter(cnt_priv, [idx], ones16)
        return carry

    lax.fori_loop(0, NCH, step, 0)
    plsc.subcore_barrier()

    base = s * RPT
    last = N - (NS - 1) * RPT

    @pl.when(s < NS - 1)
    def _():
        pltpu.sync_copy(acc_sh.at[pl.ds(base, RPT)],
                        agg_out.at[c, pl.ds(base, RPT)])

    @pl.when(s == NS - 1)
    def _():
        pltpu.sync_copy(acc_sh.at[pl.ds(base, last)],
                        agg_out.at[c, pl.ds(base, last)])

    if with_counts:
        pltpu.sync_copy(cnt_priv, cnt_out.at[c, s])


@functools.lru_cache(maxsize=None)
def _sc_kernels():
    mesh = plsc.VectorSubcoreMesh(core_axis_name="c", subcore_axis_name="s",
                                  num_cores=NC, num_subcores=NS)
    scratch_common = [
        pltpu.VMEM((NCH, CHUNK), jnp.int32),
        pltpu.VMEM((NCH, CHUNK), jnp.int32),
        pltpu.VMEM((CHUNK, D), jnp.float32),
    ]
    params = pltpu.CompilerParams(needs_layout_passes=False)
    agg_cnt = pl.kernel(
        functools.partial(_sc_body, True),
        out_type=(jax.ShapeDtypeStruct((NC, N, D), jnp.float32),
                  jax.ShapeDtypeStruct((NC, NS, ACC_ROWS), jnp.float32)),
        mesh=mesh,
        compiler_params=params,
        scratch_types=scratch_common + [
            pltpu.VMEM((ACC_ROWS,), jnp.float32),
            pltpu.VMEM_SHARED((ACC_ROWS, D), jnp.float32),
            pltpu.SemaphoreType.DMA,
        ],
    )
    agg = pl.kernel(
        functools.partial(_sc_body, False),
        out_type=jax.ShapeDtypeStruct((NC, N, D), jnp.float32),
        mesh=mesh,
        compiler_params=params,
        scratch_types=scratch_common + [
            pltpu.VMEM_SHARED((ACC_ROWS, D), jnp.float32),
            pltpu.SemaphoreType.DMA,
        ],
    )
    return agg_cnt, agg


RB = 1000


def _tc_body(relu, p_ref, cnt_ref, h_ref, wlt_ref, wrt_ref, bl_ref, out_ref):
    cnt = jnp.sum(cnt_ref[...], axis=1, keepdims=True)
    rcp = 1.0 / jnp.maximum(cnt, 1.0)
    mean = (p_ref[0] + p_ref[1]) * rcp
    out = (jnp.dot(mean, wlt_ref[...], preferred_element_type=jnp.float32)
           + jnp.dot(h_ref[...], wrt_ref[...],
                     preferred_element_type=jnp.float32)
           + bl_ref[...])
    if relu:
        out = jnp.maximum(out, 0.0)
    out_ref[...] = out


def _tc_layer(p, cntp, h, wlt, wrt, bl, relu):
    grid = (N // RB,)
    return pl.pallas_call(
        functools.partial(_tc_body, relu),
        grid=grid,
        in_specs=[
            pl.BlockSpec((NC, RB, D), lambda i: (0, i, 0)),
            pl.BlockSpec((RB, NW), lambda i: (i, 0)),
            pl.BlockSpec((RB, D), lambda i: (i, 0)),
            pl.BlockSpec((D, D), lambda i: (0, 0)),
            pl.BlockSpec((D, D), lambda i: (0, 0)),
            pl.BlockSpec((1, D), lambda i: (0, 0)),
        ],
        out_specs=pl.BlockSpec((RB, D), lambda i: (i, 0)),
        out_shape=jax.ShapeDtypeStruct((N, D), jnp.float32),
    )(p, cntp, h, wlt, wrt, bl)


def kernel(x, edge_index, Wl0, bl0, Wr0, Wl1, bl1, Wr1, Wl2, bl2, Wr2):
    src = edge_index[0]
    dst = edge_index[1]
    pad = E_PAD - E
    src_p = jnp.concatenate([src, jnp.zeros((pad,), jnp.int32)])
    dst_p = jnp.concatenate([dst, jnp.full((pad,), N, jnp.int32)])
    srcg = src_p.reshape(NW, NCH, CHUNK)
    dstg = dst_p.reshape(NW, NCH, CHUNK)
    zacc = jnp.zeros((RPT, D), jnp.float32)
    zcnt = jnp.zeros((ACC_ROWS,), jnp.float32)

    sc_agg_cnt, sc_agg = _sc_kernels()
    a0, cntp = sc_agg_cnt(x, srcg, dstg, zacc, zcnt)
    cntp = cntp.reshape(NW, ACC_ROWS).T
    h1 = _tc_layer(a0, cntp, x, Wl0.T, Wr0.T, bl0.reshape(1, D), relu=True)
    a1 = sc_agg(h1, srcg, dstg, zacc)
    h2 = _tc_layer(a1, cntp, h1, Wl1.T, Wr1.T, bl1.reshape(1, D), relu=True)
    a2 = sc_agg(h2, srcg, dstg, zacc)
    h3 = _tc_layer(a2, cntp, h2, Wl2.T, Wr2.T, bl2.reshape(1, D), relu=False)
    return h3
